# baseline (speedup 1.0000x reference)
import functools

import jax
import jax.numpy as jnp
from jax import lax
from jax.experimental import pallas as pl
from jax.experimental.pallas import tpu as pltpu
from jax.experimental.pallas import tpu_sc as plsc

_NUM_WORKERS = 32


def _sc_body(batch, seq_len, d, table_hbm, out_hbm, sem):
    c = lax.axis_index("c")
    s = lax.axis_index("s")
    wid = s * 2 + c
    rows_per_w = seq_len // _NUM_WORKERS
    base = wid * rows_per_w
    src = table_hbm.at[pl.ds(base, rows_per_w)]
    hs = []
    for b in range(batch):
        dst = out_hbm.at[b, pl.ds(base, rows_per_w)]
        hs.append(pltpu.async_copy(src, dst, sem))
    for h in hs:
        h.wait()


def kernel(x, table):
    batch, seq_len, d = x.shape
    mesh = plsc.VectorSubcoreMesh(core_axis_name="c", subcore_axis_name="s")
    k = pl.kernel(
        functools.partial(_sc_body, batch, seq_len, d),
        mesh=mesh,
        out_type=jax.ShapeDtypeStruct((batch, seq_len, d), x.dtype),
        scratch_types=[
            pltpu.SemaphoreType.DMA,
        ],
    )
    return k(table)


# speedup vs baseline: 45.3507x; 45.3507x over previous
import functools

import jax
import jax.numpy as jnp
from jax import lax
from jax.experimental import pallas as pl
from jax.experimental.pallas import tpu as pltpu
from jax.experimental.pallas import tpu_sc as plsc

_NUM_WORKERS = 32
_CHUNK_ROWS = 32
_NBUF = 3


def _sc_body(batch, seq_len, d, table_hbm, out_hbm, bufs, gsem, ssem):
    c = lax.axis_index("c")
    s = lax.axis_index("s")
    wid = c * 16 + s
    rows_per_w = seq_len // _NUM_WORKERS
    base = wid * rows_per_w
    n = rows_per_w // _CHUNK_ROWS

    def gather(i, slot):
        src = table_hbm.at[pl.ds(base + i * _CHUNK_ROWS, _CHUNK_ROWS)]
        return pltpu.async_copy(src, bufs.at[slot], gsem)

    def scatter(i, slot, b):
        dst = out_hbm.at[b, pl.ds(base + i * _CHUNK_ROWS, _CHUNK_ROWS)]
        return pltpu.async_copy(bufs.at[slot], dst, ssem)

    gh = {}
    sh = {}
    drained = set()
    gh[0] = gather(0, 0)
    if n > 1:
        gh[1] = gather(1, 1)
    for i in range(n):
        gh[i].wait()
        sh[i] = [scatter(i, i % _NBUF, b) for b in range(batch)]
        nxt = i + 2
        if nxt < n:
            prev = nxt - _NBUF
            if prev >= 0:
                for h in sh[prev]:
                    h.wait()
                drained.add(prev)
            gh[nxt] = gather(nxt, nxt % _NBUF)
    for i in range(n):
        if i not in drained:
            for h in sh[i]:
                h.wait()


def kernel(x, table):
    batch, seq_len, d = x.shape
    mesh = plsc.VectorSubcoreMesh(core_axis_name="c", subcore_axis_name="s")
    k = pl.kernel(
        functools.partial(_sc_body, batch, seq_len, d),
        mesh=mesh,
        out_type=jax.ShapeDtypeStruct((batch, seq_len, d), x.dtype),
        scratch_types=[
            pltpu.VMEM((_NBUF, _CHUNK_ROWS, d), jnp.float32),
            pltpu.SemaphoreType.DMA,
            pltpu.SemaphoreType.DMA,
        ],
    )
    return k(table)
